# baseline (speedup 1.0000x reference)
import functools

import jax
import jax.numpy as jnp
import numpy as np
from jax import lax
from jax.experimental import pallas as pl
from jax.experimental.pallas import tpu as pltpu
from jax.experimental.pallas import tpu_sc as plsc

LANES = 16
NWORKERS = 32
NBUF = 4
CHUNK = 128
DIM = 64
SEQP = 256

_PERM = np.array([32 * (p // 2) + 2 * k + (p % 2)
                  for p in range(4) for k in range(16)], dtype=np.int32)


def _widen(v32):
  vi = plsc.bitcast(v32, jnp.int32)
  lo = plsc.bitcast(jnp.left_shift(vi, 16), jnp.float32)
  hi = plsc.bitcast(jnp.bitwise_and(vi, jnp.int32(-65536)), jnp.float32)
  return lo, hi


def _sc_pool(enc_flat, emb_table, cat0, cat1, cat2, add_flat, bsz, seq):
  bpw = bsz // NWORKERS
  ngrp = DIM // LANES
  mesh = plsc.VectorSubcoreMesh(core_axis_name="c", subcore_axis_name="s")

  @functools.partial(
      pl.kernel,
      out_type=(
          jax.ShapeDtypeStruct((bsz, DIM), jnp.float32),
          jax.ShapeDtypeStruct((bsz, DIM), jnp.float32),
      ),
      mesh=mesh,
      scratch_types=[
          pltpu.VMEM((NBUF * SEQP,), jnp.int32),
          pltpu.VMEM((NBUF, seq, DIM), jnp.bfloat16),
          pltpu.VMEM((bpw, DIM), jnp.float32),
          pltpu.VMEM((3 * bpw,), jnp.int32),
          pltpu.VMEM((bpw, DIM), jnp.float32),
          pltpu.VMEM((bpw, DIM), jnp.float32),
          pltpu.VMEM((bpw, DIM), jnp.float32),
          pltpu.SemaphoreType.DMA,
          pltpu.SemaphoreType.DMA,
          pltpu.SemaphoreType.DMA,
          pltpu.SemaphoreType.DMA,
          pltpu.SemaphoreType.DMA,
          pltpu.SemaphoreType.DMA,
          pltpu.SemaphoreType.DMA,
          pltpu.SemaphoreType.DMA,
      ],
      compiler_params=pltpu.CompilerParams(use_tc_tiling_on_sc=False,
                                           needs_layout_passes=False),
  )
  def k(enc_hbm, emb_hbm, c0_hbm, c1_hbm, c2_hbm, addt_hbm,
        sums_hbm, cats_hbm,
        enc_v, rows_v, acc_v, cidx_v, ca_v, cb_v, cc_v,
        se0, se1, se2, se3, sg0, sg1, sg2, sg3):
    sems_e = (se0, se1, se2, se3)
    sems_g = (sg0, sg1, sg2, sg3)
    wid = lax.axis_index("s") * 2 + lax.axis_index("c")
    base = wid * bpw

    def enc_desc(slot, row):
      return pltpu.make_async_copy(
          enc_hbm.at[pl.ds(row * SEQP, SEQP)],
          enc_v.at[pl.ds(slot * SEQP, SEQP)], sems_e[slot])

    def gather_descs(slot):
      return (
          pltpu.make_async_copy(
              emb_hbm.at[enc_v.at[pl.ds(slot * SEQP, CHUNK)]],
              rows_v.at[slot, pl.ds(0, CHUNK)], sems_g[slot]),
          pltpu.make_async_copy(
              emb_hbm.at[enc_v.at[pl.ds(slot * SEQP + CHUNK, seq - CHUNK)]],
              rows_v.at[slot, pl.ds(CHUNK, seq - CHUNK)], sems_g[slot]),
      )

    def fire(slot, row):
      enc_desc(slot, row).wait()
      for d in gather_descs(slot):
        d.start()

    for s in range(NBUF):
      enc_desc(s, base + s).start()
    fire(0, base)
    fire(1, base + 1)

    for j, (tab, dst) in enumerate(
        ((c0_hbm, ca_v), (c1_hbm, cb_v), (c2_hbm, cc_v))):
      pltpu.sync_copy(addt_hbm.at[pl.ds(j * bsz + base, bpw)],
                      cidx_v.at[pl.ds(j * bpw, bpw)])
      pltpu.sync_copy(tab.at[cidx_v.at[pl.ds(j * bpw, bpw)]], dst)

    @pl.loop(0, bpw, unroll=4)
    def _(b):
      for g in range(ngrp):
        sl = pl.ds(g * LANES, LANES)
        ca_v[b, sl] = ca_v[b, sl] + cb_v[b, sl] + cc_v[b, sl]

    pltpu.sync_copy(ca_v, cats_hbm.at[pl.ds(base, bpw)])

    @pl.loop(0, bpw // NBUF)
    def _(i):
      for s in range(NBUF):
        b_local = i * NBUF + s

        def stage():
          fire((s + 2) % NBUF, base + b_local + 2)
        if s < 2:
          stage()
        else:
          pl.when(i < bpw // NBUF - 1)(stage)

        for d in gather_descs(s):
          d.wait()

        zeros = (jnp.zeros((LANES,), jnp.float32),) * ngrp

        @pl.loop(0, seq, init_carry=zeros, unroll=8)
        def totals(t, carry):
          acc = list(carry)
          for g in range(2):
            lo, hi = _widen(rows_v[s, t, pl.ds(g * 2 * LANES, 2 * LANES)])
            acc[2 * g] = acc[2 * g] + lo
            acc[2 * g + 1] = acc[2 * g + 1] + hi
          return tuple(acc)

        for g in range(ngrp):
          acc_v[b_local, pl.ds(g * LANES, LANES)] = totals[g]

        def refill():
          enc_desc(s, base + b_local + NBUF).start()
        pl.when(i < bpw // NBUF - 1)(refill)

    pltpu.sync_copy(acc_v, sums_hbm.at[pl.ds(base, bpw)])

  return k(enc_flat, emb_table, cat0, cat1, cat2, add_flat)


def _tc_head(sums, cats, encoded_text, w_t, bias):
  bsz, seq = encoded_text.shape
  dim = sums.shape[1]
  ncls = w_t.shape[1]
  blk = 256

  def body(sums_ref, cats_ref, enc_ref, wt_ref, b_ref, out_ref):
    cnt = jnp.sum((enc_ref[...] != 0).astype(jnp.float32), axis=1,
                  keepdims=True)
    x = jnp.where(cnt > 0.0, sums_ref[...] / cnt, 0.0)
    x = x + cats_ref[...]
    z = lax.dot_general(x, wt_ref[...], (((1,), (0,)), ((), ())),
                        preferred_element_type=jnp.float32)
    out_ref[...] = z + b_ref[...]

  return pl.pallas_call(
      body,
      grid=(bsz // blk,),
      in_specs=[
          pl.BlockSpec((blk, dim), lambda i: (i, 0)),
          pl.BlockSpec((blk, dim), lambda i: (i, 0)),
          pl.BlockSpec((blk, seq), lambda i: (i, 0)),
          pl.BlockSpec((dim, ncls), lambda i: (0, 0)),
          pl.BlockSpec((1, ncls), lambda i: (0, 0)),
      ],
      out_specs=pl.BlockSpec((blk, ncls), lambda i: (i, 0)),
      out_shape=jax.ShapeDtypeStruct((bsz, ncls), jnp.float32),
  )(sums, cats, encoded_text, w_t, bias)


def kernel(encoded_text, additional_inputs, emb_table, cat0, cat1, cat2, W, b):
  bsz, seq = encoded_text.shape
  enc_flat = jnp.pad(encoded_text, ((0, 0), (0, SEQP - seq))).reshape(-1)
  add_flat = additional_inputs.T.reshape(-1)
  sums, cats = _sc_pool(enc_flat, emb_table.astype(jnp.bfloat16),
                        cat0, cat1, cat2, add_flat, bsz, seq)
  cats_p = jnp.take(cats, _PERM, axis=1)
  wt_p = jnp.take(W.T, _PERM, axis=0)
  return _tc_head(sums, cats_p, encoded_text, wt_p, b.reshape(1, -1))


# speedup vs baseline: 1.2788x; 1.2788x over previous
import functools

import jax
import jax.numpy as jnp
from jax import lax
from jax.experimental import pallas as pl
from jax.experimental.pallas import tpu as pltpu
from jax.experimental.pallas import tpu_sc as plsc

LANES = 16
NWORKERS = 32
NBUF = 4
CHUNK = 128
DIM = 64
SEQP = 256


def _sc_pool(enc_flat, emb_table, cat0, cat1, cat2, add_flat, bsz, seq):
  bpw = bsz // NWORKERS
  ngrp = DIM // LANES
  mesh = plsc.VectorSubcoreMesh(core_axis_name="c", subcore_axis_name="s")

  @functools.partial(
      pl.kernel,
      out_type=(
          jax.ShapeDtypeStruct((bsz, DIM), jnp.float32),
          jax.ShapeDtypeStruct((bsz, DIM), jnp.float32),
      ),
      mesh=mesh,
      scratch_types=[
          pltpu.VMEM((NBUF * SEQP,), jnp.int32),
          pltpu.VMEM((NBUF, seq, DIM), jnp.float32),
          pltpu.VMEM((bpw, DIM), jnp.float32),
          pltpu.VMEM((3 * bpw,), jnp.int32),
          pltpu.VMEM((bpw, DIM), jnp.float32),
          pltpu.VMEM((bpw, DIM), jnp.float32),
          pltpu.VMEM((bpw, DIM), jnp.float32),
          pltpu.SemaphoreType.DMA,
          pltpu.SemaphoreType.DMA,
          pltpu.SemaphoreType.DMA,
          pltpu.SemaphoreType.DMA,
          pltpu.SemaphoreType.DMA,
          pltpu.SemaphoreType.DMA,
          pltpu.SemaphoreType.DMA,
          pltpu.SemaphoreType.DMA,
      ],
      compiler_params=pltpu.CompilerParams(use_tc_tiling_on_sc=False),
  )
  def k(enc_hbm, emb_hbm, c0_hbm, c1_hbm, c2_hbm, addt_hbm,
        sums_hbm, cats_hbm,
        enc_v, rows_v, acc_v, cidx_v, ca_v, cb_v, cc_v,
        se0, se1, se2, se3, sg0, sg1, sg2, sg3):
    sems_e = (se0, se1, se2, se3)
    sems_g = (sg0, sg1, sg2, sg3)
    wid = lax.axis_index("s") * 2 + lax.axis_index("c")
    base = wid * bpw

    def enc_desc(slot, row):
      return pltpu.make_async_copy(
          enc_hbm.at[pl.ds(row * SEQP, SEQP)],
          enc_v.at[pl.ds(slot * SEQP, SEQP)], sems_e[slot])

    def gather_descs(slot):
      return (
          pltpu.make_async_copy(
              emb_hbm.at[enc_v.at[pl.ds(slot * SEQP, CHUNK)]],
              rows_v.at[slot, pl.ds(0, CHUNK)], sems_g[slot]),
          pltpu.make_async_copy(
              emb_hbm.at[enc_v.at[pl.ds(slot * SEQP + CHUNK, seq - CHUNK)]],
              rows_v.at[slot, pl.ds(CHUNK, seq - CHUNK)], sems_g[slot]),
      )

    def fire(slot, row):
      enc_desc(slot, row).wait()
      for d in gather_descs(slot):
        d.start()

    for s in range(NBUF):
      enc_desc(s, base + s).start()
    fire(0, base)
    fire(1, base + 1)

    for j, (tab, dst) in enumerate(
        ((c0_hbm, ca_v), (c1_hbm, cb_v), (c2_hbm, cc_v))):
      pltpu.sync_copy(addt_hbm.at[pl.ds(j * bsz + base, bpw)],
                      cidx_v.at[pl.ds(j * bpw, bpw)])
      pltpu.sync_copy(tab.at[cidx_v.at[pl.ds(j * bpw, bpw)]], dst)

    @pl.loop(0, bpw, unroll=4)
    def _(b):
      for g in range(ngrp):
        sl = pl.ds(g * LANES, LANES)
        ca_v[b, sl] = ca_v[b, sl] + cb_v[b, sl] + cc_v[b, sl]

    pltpu.sync_copy(ca_v, cats_hbm.at[pl.ds(base, bpw)])

    @pl.loop(0, bpw // NBUF)
    def _(i):
      for s in range(NBUF):
        b_local = i * NBUF + s

        def stage():
          fire((s + 2) % NBUF, base + b_local + 2)
        if s < 2:
          stage()
        else:
          pl.when(i < bpw // NBUF - 1)(stage)

        for d in gather_descs(s):
          d.wait()

        zeros = (jnp.zeros((LANES,), jnp.float32),) * ngrp

        @pl.loop(0, seq, init_carry=zeros, unroll=8)
        def totals(t, carry):
          return tuple(
              c + rows_v[s, t, pl.ds(g * LANES, LANES)]
              for g, c in enumerate(carry))

        for g in range(ngrp):
          acc_v[b_local, pl.ds(g * LANES, LANES)] = totals[g]

        def refill():
          enc_desc(s, base + b_local + NBUF).start()
        pl.when(i < bpw // NBUF - 1)(refill)

    pltpu.sync_copy(acc_v, sums_hbm.at[pl.ds(base, bpw)])

  return k(enc_flat, emb_table, cat0, cat1, cat2, add_flat)


def _tc_head(sums, cats, encoded_text, w_t, bias):
  bsz, seq = encoded_text.shape
  dim = sums.shape[1]
  ncls = w_t.shape[1]
  blk = 256

  def body(sums_ref, cats_ref, enc_ref, wt_ref, b_ref, out_ref):
    cnt = jnp.sum((enc_ref[...] != 0).astype(jnp.float32), axis=1,
                  keepdims=True)
    x = jnp.where(cnt > 0.0, sums_ref[...] / cnt, 0.0)
    x = x + cats_ref[...]
    z = lax.dot_general(x, wt_ref[...], (((1,), (0,)), ((), ())),
                        preferred_element_type=jnp.float32)
    out_ref[...] = z + b_ref[...]

  return pl.pallas_call(
      body,
      grid=(bsz // blk,),
      in_specs=[
          pl.BlockSpec((blk, dim), lambda i: (i, 0)),
          pl.BlockSpec((blk, dim), lambda i: (i, 0)),
          pl.BlockSpec((blk, seq), lambda i: (i, 0)),
          pl.BlockSpec((dim, ncls), lambda i: (0, 0)),
          pl.BlockSpec((1, ncls), lambda i: (0, 0)),
      ],
      out_specs=pl.BlockSpec((blk, ncls), lambda i: (i, 0)),
      out_shape=jax.ShapeDtypeStruct((bsz, ncls), jnp.float32),
  )(sums, cats, encoded_text, w_t, bias)


def kernel(encoded_text, additional_inputs, emb_table, cat0, cat1, cat2, W, b):
  bsz, seq = encoded_text.shape
  enc_flat = jnp.pad(encoded_text, ((0, 0), (0, SEQP - seq))).reshape(-1)
  add_flat = additional_inputs.T.reshape(-1)
  sums, cats = _sc_pool(enc_flat, emb_table, cat0, cat1, cat2, add_flat,
                        bsz, seq)
  return _tc_head(sums, cats, encoded_text, W.T, b.reshape(1, -1))
